# Initial kernel scaffold; baseline (speedup 1.0000x reference)
import jax
import jax.numpy as jnp
from jax.experimental import pallas as pl


def kernel(x, edge_index, etypes, W0, b0, W1, b1, W2, b2, gate_w, gate_b, fc1_w, fc1_b, fc2_w, fc2_b, fc3_w, fc3_b):
    raise NotImplementedError("write your pallas kernel here")


# speedup vs baseline: 18.4176x; 18.4176x over previous
import functools

import jax
import jax.numpy as jnp
from jax import lax
from jax.experimental import pallas as pl
from jax.experimental.pallas import tpu as pltpu
from jax.experimental.pallas import tpu_sc as plsc

N_NODES = 10000
N_PAD = 10240
DUMMY_ROW = 10200
E_EDGES = 320000
K_CHUNK = 128
N_TILES = 32
CHUNKS_PER_TILE = 80
E_PAD = N_TILES * CHUNKS_PER_TILE * K_CHUNK
D_FEAT = 128
H_FEAT = 128
R_REL = 4
BN = 1024


def _mm_body(p0_ref, p1_ref, b_ref, w_ref, out_ref):
    hin = p0_ref[...] + p1_ref[...] + b_ref[...]
    for r in range(R_REL):
        out_ref[r] = jnp.dot(hin, w_ref[r], preferred_element_type=jnp.float32)


_rgcn_mm = pl.pallas_call(
    _mm_body,
    grid=(N_PAD // BN,),
    in_specs=[
        pl.BlockSpec((BN, D_FEAT), lambda i: (i, 0)),
        pl.BlockSpec((BN, D_FEAT), lambda i: (i, 0)),
        pl.BlockSpec((1, H_FEAT), lambda i: (0, 0)),
        pl.BlockSpec((R_REL, D_FEAT, H_FEAT), lambda i: (0, 0, 0)),
    ],
    out_specs=pl.BlockSpec((R_REL, BN, H_FEAT), lambda i: (0, i, 0)),
    out_shape=jax.ShapeDtypeStruct((R_REL, N_PAD, H_FEAT), jnp.float32),
)


_ROWS_PER_TILE = N_PAD // 16


_HALF_CHUNKS = CHUNKS_PER_TILE // 2


def _sc_aggregate_body(y_hbm, g_hbm, d_hbm, zeros_hbm, out_hbm,
                       g_v, d_v, rows_v, acc_sh, sem0, sem1):
    cid = lax.axis_index("c")
    sid = lax.axis_index("s")
    wid = cid * 16 + sid

    pltpu.sync_copy(zeros_hbm.at[pl.ds(sid * _ROWS_PER_TILE, _ROWS_PER_TILE)],
                    acc_sh.at[pl.ds(sid * _ROWS_PER_TILE, _ROWS_PER_TILE)])
    plsc.subcore_barrier()

    sems = (sem0, sem1)

    def start(j, buf, sem):
        pltpu.async_copy(y_hbm.at[g_v.at[j]], rows_v.at[buf], sem)

    def wait(j, buf, sem):
        pltpu.make_async_copy(y_hbm.at[g_v.at[j]], rows_v.at[buf], sem).wait()

    def scatter(j, buf):
        pltpu.sync_copy(rows_v.at[buf], acc_sh.at[d_v.at[j]], add=True)

    for half in range(2):
        pltpu.sync_copy(g_hbm.at[wid * 2 + half], g_v)
        pltpu.sync_copy(d_hbm.at[wid * 2 + half], d_v)

        start(0, 0, sem0)
        start(1, 1, sem1)

        def loop_body(jj, carry):
            for b in range(2):
                j = 2 * jj + b
                wait(j, b, sems[b])
                scatter(j, b)
                start(j + 2, b, sems[b])
            return carry

        lax.fori_loop(0, _HALF_CHUNKS // 2 - 1, loop_body, 0)
        for b in range(2):
            j = _HALF_CHUNKS - 2 + b
            wait(j, b, sems[b])
            scatter(j, b)

    plsc.subcore_barrier()
    base = cid * N_PAD + sid * _ROWS_PER_TILE
    pltpu.sync_copy(acc_sh.at[pl.ds(sid * _ROWS_PER_TILE, _ROWS_PER_TILE)],
                    out_hbm.at[pl.ds(base, _ROWS_PER_TILE)])


@functools.cache
def _get_sc_aggregate():
    return pl.kernel(
        _sc_aggregate_body,
        out_type=jax.ShapeDtypeStruct((2 * N_PAD, H_FEAT), jnp.float32),
        mesh=plsc.VectorSubcoreMesh(core_axis_name="c", subcore_axis_name="s"),
        scratch_types=[
            pltpu.VMEM((_HALF_CHUNKS, K_CHUNK), jnp.int32),
            pltpu.VMEM((_HALF_CHUNKS, K_CHUNK), jnp.int32),
            pltpu.VMEM((2, K_CHUNK, H_FEAT), jnp.float32),
            pltpu.VMEM_SHARED((N_PAD, H_FEAT), jnp.float32),
            pltpu.SemaphoreType.DMA,
            pltpu.SemaphoreType.DMA,
        ],
    )


def _head_body(p0_ref, p1_ref, b_ref, gw_ref, gb_ref,
               fc1w_ref, fc1b_ref, fc2w_ref, fc2b_ref, fc3w_ref, fc3b_ref,
               out_ref):
    h3 = p0_ref[...] + p1_ref[...] + b_ref[...]
    rows = lax.broadcasted_iota(jnp.int32, (N_PAD, 1), 0)
    valid = rows < N_NODES
    logit = jnp.sum(h3 * gw_ref[...], axis=1, keepdims=True) + gb_ref[0, 0]
    logit = jnp.where(valid, logit, jnp.float32(-1e30))
    m = jnp.max(logit)
    e = jnp.where(valid, jnp.exp(logit - m), jnp.float32(0.0))
    gate = e / jnp.sum(e)
    readout = jnp.sum(h3 * gate, axis=0, keepdims=True)
    z = jnp.dot(readout, fc1w_ref[...], preferred_element_type=jnp.float32)
    z = jnp.maximum(z + fc1b_ref[...], 0.0)
    z = jnp.dot(z, fc2w_ref[...], preferred_element_type=jnp.float32)
    z = jnp.maximum(z + fc2b_ref[...], 0.0)
    z = jnp.dot(z, fc3w_ref[...], preferred_element_type=jnp.float32)
    z = z + fc3b_ref[...]
    out_ref[...] = 1.0 / (1.0 + jnp.exp(-z[:, :1]))


_head = pl.pallas_call(
    _head_body,
    out_shape=jax.ShapeDtypeStruct((1, 1), jnp.float32),
)


def _pad2(a, rows, cols):
    return jnp.pad(a, ((0, rows - a.shape[0]), (0, cols - a.shape[1])))


def kernel(x, edge_index, etypes, W0, b0, W1, b1, W2, b2,
           gate_w, gate_b, fc1_w, fc1_b, fc2_w, fc2_b, fc3_w, fc3_b):
    src = edge_index[0]
    dst = edge_index[1]
    g = etypes * N_PAD + src
    g = jnp.concatenate([g, jnp.zeros((E_PAD - E_EDGES,), jnp.int32)])
    d = jnp.concatenate(
        [dst, jnp.full((E_PAD - E_EDGES,), DUMMY_ROW, jnp.int32)])
    g3 = g.reshape(N_TILES * 2, CHUNKS_PER_TILE // 2, K_CHUNK)
    d3 = d.reshape(N_TILES * 2, CHUNKS_PER_TILE // 2, K_CHUNK)

    zeros_nh = jnp.zeros((N_PAD, H_FEAT), jnp.float32)
    zero_b = jnp.zeros((1, H_FEAT), jnp.float32)
    xpad = jnp.pad(x, ((0, N_PAD - N_NODES), (0, 0)))

    sc_aggregate = _get_sc_aggregate()

    def layer(p0, p1, bprev, W):
        y = _rgcn_mm(p0, p1, bprev, W).reshape(R_REL * N_PAD, H_FEAT)
        part = sc_aggregate(y, g3, d3, zeros_nh)
        return part[:N_PAD], part[N_PAD:]

    p0, p1 = layer(xpad, zeros_nh, zero_b, W0)
    p0, p1 = layer(p0, p1, b0.reshape(1, H_FEAT), W1)
    p0, p1 = layer(p0, p1, b1.reshape(1, H_FEAT), W2)

    out = _head(
        p0, p1, b2.reshape(1, H_FEAT),
        gate_w.reshape(1, H_FEAT),
        gate_b.reshape(1, 1),
        _pad2(fc1_w, 128, 128), _pad2(fc1_b.reshape(1, -1), 1, 128),
        _pad2(fc2_w, 128, 128), _pad2(fc2_b.reshape(1, -1), 1, 128),
        _pad2(fc3_w, 128, 128), _pad2(fc3_b.reshape(1, -1), 1, 128),
    )
    return out.reshape((1,))
